# baseline (speedup 1.0000x reference)
import functools

import jax
import jax.numpy as jnp
from jax import lax
from jax.experimental import pallas as pl
from jax.experimental.pallas import tpu as pltpu
from jax.experimental.pallas import tpu_sc as plsc

M = 100000
D = 512
B = 16384
L = 16
NC = 2
NS = 16
NW = NC * NS
RPW = B // NW
R = 8
NCHUNK = RPW // R
NBUF = 4
IC = 2048
NIC = B // IC
SG = 4


@functools.cache
def _build():
    mesh = plsc.VectorSubcoreMesh(core_axis_name="c", subcore_axis_name="s")
    return functools.partial(
        pl.kernel,
        mesh=mesh,
        compiler_params=pltpu.CompilerParams(needs_layout_passes=False),
        out_type=jax.ShapeDtypeStruct((B, D), jnp.float32),
        scratch_types=[
            pltpu.VMEM((M,), jnp.int32),
            pltpu.VMEM((2, IC), jnp.int32),
            pltpu.VMEM((RPW,), jnp.int32),
            pltpu.VMEM((RPW,), jnp.int32),
            pltpu.VMEM((NBUF, R, D), jnp.float32),
            pltpu.SemaphoreType.DMA((2,)),
            pltpu.SemaphoreType.DMA((NBUF,)),
            pltpu.SemaphoreType.DMA((NBUF,)),
        ],
    )(_scatter_read)


def _scatter_read(idx_hbm, val_hbm, out_hbm, win, idxc, own_idx, wbuf, rows,
                  isems, gsems, osems):
    wid = lax.axis_index("s") * NC + lax.axis_index("c")
    obase = wid * RPW

    lane = lax.iota(jnp.int32, L)

    def idx_load(ci, slot):
        return pltpu.make_async_copy(
            idx_hbm.at[pl.ds(ci * IC, IC)], idxc.at[slot], isems.at[slot])

    idx_load(0, 0).start()
    idx_load(1, 1).start()

    for ci in range(NIC):
        slot = ci % 2
        idx_load(ci, slot).wait()

        def scan_body(g, carry, _s=slot, _b=ci * IC):
            ivs = [idxc[_s, pl.ds((g * SG + u) * L, L)] for u in range(SG)]
            for u in range(SG):
                jv = _b + (g * SG + u) * L + lane
                plsc.store_scatter(win, [ivs[u]], jv)
            return carry

        lax.fori_loop(0, IC // L // SG, scan_body, 0, unroll=2)
        if ci + 2 < NIC:
            idx_load(ci + 2, slot).start()

    pltpu.sync_copy(idx_hbm.at[pl.ds(obase, RPW)], own_idx)

    def wloop(g, carry):
        ivs = [own_idx[pl.ds((g * SG + u) * L, L)] for u in range(SG)]
        ws = [plsc.load_gather(win, [ivs[u]]) for u in range(SG)]
        for u in range(SG):
            wbuf[pl.ds((g * SG + u) * L, L)] = ws[u]
        return carry

    lax.fori_loop(0, RPW // L // SG, wloop, 0, unroll=2)

    def gather(k, slot):
        return pltpu.make_async_copy(
            val_hbm.at[wbuf.at[pl.ds(k * R, R)]], rows.at[slot],
            gsems.at[slot])

    def put(k, slot):
        return pltpu.make_async_copy(
            rows.at[slot], out_hbm.at[pl.ds(obase + k * R, R)],
            osems.at[slot])

    for k in range(NBUF - 1):
        gather(k, k).start()

    for k in range(NCHUNK):
        nxt = k + NBUF - 1
        nslot = nxt % NBUF
        if nxt < NCHUNK:
            if nxt >= NBUF:
                put(nxt - NBUF, nslot).wait()
            gather(nxt, nslot).start()
        gather(k, k % NBUF).wait()
        put(k, k % NBUF).start()

    for k in range(NCHUNK - NBUF, NCHUNK):
        put(k, k % NBUF).wait()


def kernel(mem, indices, values):
    del mem
    idx = indices.astype(jnp.int32)
    return _build()(idx, values)


# speedup vs baseline: 1.0441x; 1.0441x over previous
import functools

import jax
import jax.numpy as jnp
from jax import lax
from jax.experimental import pallas as pl
from jax.experimental.pallas import tpu as pltpu
from jax.experimental.pallas import tpu_sc as plsc

M = 100000
D = 512
B = 16384
L = 16
NC = 2
NS = 16
NW = NC * NS
RPW = B // NW
R = 8
NCHUNK = RPW // R
NBUF = 4
IC = 2048
NIC = B // IC
SG = 4


@functools.cache
def _build():
    mesh = plsc.VectorSubcoreMesh(core_axis_name="c", subcore_axis_name="s")
    return functools.partial(
        pl.kernel,
        mesh=mesh,
        compiler_params=pltpu.CompilerParams(needs_layout_passes=False),
        out_type=jax.ShapeDtypeStruct((B, D), jnp.float32),
        scratch_types=[
            pltpu.VMEM((M,), jnp.int32),
            pltpu.VMEM((2, IC), jnp.int32),
            pltpu.VMEM((RPW,), jnp.int32),
            pltpu.VMEM((RPW,), jnp.int32),
            pltpu.VMEM((NBUF, R, D), jnp.float32),
            pltpu.SemaphoreType.DMA((2,)),
            pltpu.SemaphoreType.DMA((NBUF,)),
            pltpu.SemaphoreType.DMA((NBUF,)),
        ],
    )(_scatter_read)


def _scatter_read(idx_hbm, val_hbm, out_hbm, win, idxc, own_idx, wbuf, rows,
                  isems, gsems, osems):
    wid = lax.axis_index("s") * NC + lax.axis_index("c")
    obase = wid * RPW

    lane = lax.iota(jnp.int32, L)

    def idx_load(ci, slot):
        return pltpu.make_async_copy(
            idx_hbm.at[pl.ds(ci * IC, IC)], idxc.at[slot], isems.at[slot])

    idx_load(0, 0).start()
    idx_load(1, 1).start()

    for ci in range(NIC):
        slot = ci % 2
        idx_load(ci, slot).wait()

        def scan_body(g, carry, _s=slot, _b=ci * IC):
            ivs = [idxc[_s, pl.ds((g * SG + u) * L, L)] for u in range(SG)]
            for u in range(SG):
                jv = _b + (g * SG + u) * L + lane
                plsc.store_scatter(win, [ivs[u]], jv)
            return carry

        lax.fori_loop(0, IC // L // SG, scan_body, 0, unroll=2)
        if ci + 2 < NIC:
            idx_load(ci + 2, slot).start()

    pltpu.sync_copy(idx_hbm.at[pl.ds(obase, RPW)], own_idx)

    def wloop(g, carry):
        ivs = [own_idx[pl.ds((g * SG + u) * L, L)] for u in range(SG)]
        ws = [plsc.load_gather(win, [ivs[u]]) for u in range(SG)]
        for u in range(SG):
            wbuf[pl.ds((g * SG + u) * L, L)] = ws[u]
        return carry

    lax.fori_loop(0, RPW // L // SG, wloop, 0, unroll=2)

    def gather(k, slot):
        return pltpu.make_async_copy(
            val_hbm.at[wbuf.at[pl.ds(k * R, R)]], rows.at[slot],
            gsems.at[slot])

    def put(k, slot):
        return pltpu.make_async_copy(
            rows.at[slot], out_hbm.at[pl.ds(obase + k * R, R)],
            osems.at[slot])

    for k in range(NBUF - 1):
        gather(k, k).start()

    def chunk_quad(t, carry):
        for u in range(NBUF):
            k = t * NBUF + u
            nxt = k + NBUF - 1
            nslot = (u + NBUF - 1) % NBUF

            @pl.when(nxt < NCHUNK)
            def _():
                @pl.when(nxt >= NBUF)
                def _():
                    put(nxt - NBUF, nslot).wait()

                gather(nxt, nslot).start()

            gather(k, u).wait()
            put(k, u).start()
        return carry

    lax.fori_loop(0, NCHUNK // NBUF, chunk_quad, 0)

    for k in range(NCHUNK - NBUF, NCHUNK):
        put(k, k % NBUF).wait()


def kernel(mem, indices, values):
    del mem
    idx = indices.astype(jnp.int32)
    return _build()(idx, values)
